# Initial kernel scaffold; baseline (speedup 1.0000x reference)
import jax
import jax.numpy as jnp
from jax.experimental import pallas as pl


def kernel(x, edge_index, rows1, cols1, rows2, cols2, rows3, cols3, w1_vals, b1, w2_vals, b2, w3_vals, b3, g1, beta1, g2, beta2):
    raise NotImplementedError("write your pallas kernel here")


# speedup vs baseline: 39.3517x; 39.3517x over previous
import functools

import jax
import jax.numpy as jnp
from jax import lax
from jax.experimental import pallas as pl
from jax.experimental.pallas import tpu as pltpu
from jax.experimental.pallas import tpu_sc as plsc

_LANES = 16
_CH = 128
_CHN = 16
_L = 4
_N_OUT = 1000


def _elu(v):
    return jnp.where(v > 0, v, jnp.exp(v) - 1.0)


def _rsqrt_vec(v):
    i = lax.bitcast_convert_type(v, jnp.int32)
    i = jnp.full((_LANES,), 0x5F3759DF, jnp.int32) - (i >> 1)
    y = lax.bitcast_convert_type(i, jnp.float32)
    for _ in range(3):
        y = y * (1.5 - 0.5 * v * y * y)
    return y


def _bn_elu_row(z, g, b, mu, var):
    inv = _rsqrt_vec(jnp.full((_LANES,), var + 1e-5, jnp.float32))
    a = inv * g
    return _elu(z * a + (b - mu * a))


def _row_stats(z):
    mu = jnp.sum(z) * (1.0 / _LANES)
    var = jnp.sum(z * z) * (1.0 / _LANES) - mu * mu
    return mu, var


def kernel(x, edge_index, rows1, cols1, rows2, cols2, rows3, cols3,
           w1_vals, b1, w2_vals, b2, w3_vals, b3, g1, beta1, g2, beta2):
    B, N = x.shape
    E = edge_index.shape[1]
    C = rows1.shape[0] // E
    F = rows2.shape[0] // (C * C)
    Ne3 = rows3.shape[0] // C
    H = N * C
    assert B == _LANES and E % _CH == 0 and N % _CHN == 0

    src = edge_index[0]
    dst = edge_index[1]
    x_t = x.T
    w1r = w1_vals
    w2r = w2_vals
    w3r = w3_vals.reshape(Ne3, C)
    e3src = rows3.reshape(Ne3, C)[:, 0] // C
    e3ids = cols3.reshape(Ne3, C)[:, 0]
    Ne3p = ((Ne3 + _CH - 1) // _CH) * _CH
    padk = Ne3p - Ne3
    e3src_p = jnp.concatenate([e3src, jnp.full((padk,), N, jnp.int32)])
    e3ids_p = jnp.concatenate([e3ids, jnp.full((padk,), E, jnp.int32)])
    w3r_p = jnp.concatenate([w3r, jnp.zeros((padk, C), jnp.float32)]).reshape(-1)
    nfn0_vec = jnp.full((_LANES,), rows2[0] // C, jnp.int32)

    NEC = E // _CH
    NNC = N // _CHN
    NE3C = Ne3p // _CH
    NZC = N // _CHN + 1
    NOC = _N_OUT // 8
    NS = 16
    NONR = ((N + 16 + _CH - 1) // _CH) * _CH

    def it_count(total):
        return (total + NS - 1) // NS

    mesh = plsc.VectorSubcoreMesh(core_axis_name="c", subcore_axis_name="s")

    @functools.partial(
        pl.kernel,
        out_type=(
            jax.ShapeDtypeStruct((_N_OUT, _LANES), jnp.float32),
            jax.ShapeDtypeStruct((E + 8, _LANES), jnp.float32),
            jax.ShapeDtypeStruct((E + 8, _LANES), jnp.float32),
        ),
        mesh=mesh,
        compiler_params=pltpu.CompilerParams(needs_layout_passes=False,
                                             use_tc_tiling_on_sc=False),
        scratch_types=(
            pltpu.VMEM_SHARED((N + 16, C, _LANES), jnp.float32),
            pltpu.VMEM_SHARED((NONR, _LANES), jnp.float32),
            pltpu.VMEM((_CH, _LANES), jnp.float32),
            pltpu.VMEM((_CH, C, _LANES), jnp.float32),
            pltpu.VMEM((_CH,), jnp.int32),
            pltpu.VMEM((_CH,), jnp.int32),
            pltpu.VMEM((_CH * 8 + 16,), jnp.float32),
            pltpu.VMEM((_CH * 8 + 16,), jnp.float32),
            pltpu.VMEM((4, _CH + 16), jnp.float32),
            pltpu.VMEM((_CHN, C, _LANES), jnp.float32),
            pltpu.VMEM((_CH, _LANES), jnp.float32),
            pltpu.VMEM((8, _LANES), jnp.float32),
            pltpu.VMEM((_LANES,), jnp.int32),
            pltpu.SemaphoreType.DMA,
            pltpu.SemaphoreType.DMA,
        ),
    )
    def gsnn(xt_h, src_h, dst_h, w1_h, w2_h, g1_h, b1g_h, g2_h, b2g_h,
             w3_h, e3s_h, e3i_h, b3_h, nfn_h,
             out_h, h_h, x0b_h,
             zsh, onsh, hbuf, stage, idxa, idxb, wbuf, w2b, gbuf, zbuf, zrow,
             obuf, pvec, sem1, sem2):
        s = lax.axis_index("s")
        zero16 = jnp.zeros((_LANES,), jnp.float32)

        pltpu.sync_copy(nfn_h, pvec)
        nfn0 = pvec[...][0]

        def zfill(i, _):
            zrow[i, :] = zero16
            return 0
        lax.fori_loop(0, _CH, zfill, 0)

        def zfill2(i, _):
            for c in range(C):
                zbuf[i, c, :] = zero16
            return 0
        lax.fori_loop(0, _CHN, zfill2, 0)

        def zeroz(it, _):
            ch = it * NS + s
            @pl.when(ch < NZC)
            def _():
                pltpu.sync_copy(zbuf, zsh.at[pl.ds(ch * _CHN, _CHN)])
            return 0
        lax.fori_loop(0, it_count(NZC), zeroz, 0)

        def zeroon(it, _):
            ch = it * NS + s
            @pl.when(ch < NONR // _CH)
            def _():
                pltpu.sync_copy(zrow, onsh.at[pl.ds(ch * _CH, _CH)])
            return 0
        lax.fori_loop(0, it_count(NONR // _CH), zeroon, 0)

        def ph0(it, _):
            ch = it * NS + s
            @pl.when(ch < NEC)
            def _():
                e0 = ch * _CH
                pltpu.sync_copy(src_h.at[pl.ds(e0, _CH)], idxa)
                pltpu.async_copy(xt_h.at[idxa], hbuf, sem1).wait()
                pltpu.sync_copy(hbuf, h_h.at[pl.ds(e0, _CH)])
                pltpu.sync_copy(b3_h.at[pl.ds(e0, _CH)],
                                gbuf.at[0, pl.ds(0, _CH)])

                @plsc.parallel_loop(0, _CH, 1, unroll=4)
                def badd(i):
                    bv = gbuf[0, pl.ds(i, _LANES)]
                    hbuf[i, :] = hbuf[i, :] + bv[0]
                pltpu.sync_copy(hbuf, x0b_h.at[pl.ds(e0, _CH)])
            return 0
        lax.fori_loop(0, it_count(NEC), ph0, 0)
        plsc.subcore_barrier()

        for layer in range(_L):
            def pha(it, _):
                ch = it * NS + s
                @pl.when(ch < NEC)
                def _():
                    e0 = ch * _CH
                    pltpu.sync_copy(h_h.at[pl.ds(e0, _CH)], hbuf)
                    pltpu.sync_copy(dst_h.at[pl.ds(e0, _CH)], idxa)
                    pltpu.sync_copy(w1_h.at[pl.ds(e0 * C, _CH * C)],
                                    wbuf.at[pl.ds(0, _CH * C)])

                    @plsc.parallel_loop(0, _CH, 1, unroll=4)
                    def edge(i):
                        hrow = hbuf[i, :]
                        wv = wbuf[pl.ds(i * C, _LANES)]
                        for c in range(C):
                            stage[i, c, :] = hrow * wv[c]
                    pltpu.sync_copy(stage, zsh.at[idxa], add=True)
                return 0
            lax.fori_loop(0, it_count(NEC), pha, 0)
            plsc.subcore_barrier()

            def phb(it, _):
                ch = it * NS + s
                @pl.when(ch < NNC)
                def _():
                    n0 = ch * _CHN
                    r0 = n0 * C
                    pltpu.sync_copy(zsh.at[pl.ds(n0, _CHN)], zbuf)
                    pltpu.sync_copy(g1_h.at[pl.ds(r0, _CH)],
                                    gbuf.at[0, pl.ds(0, _CH)])
                    pltpu.sync_copy(b1g_h.at[pl.ds(r0, _CH)],
                                    gbuf.at[1, pl.ds(0, _CH)])
                    pltpu.sync_copy(g2_h.at[pl.ds(r0, _CH)],
                                    gbuf.at[2, pl.ds(0, _CH)])
                    pltpu.sync_copy(b2g_h.at[pl.ds(r0, _CH)],
                                    gbuf.at[3, pl.ds(0, _CH)])
                    w2start = jnp.clip(n0 - nfn0, 0, F - _CHN)
                    pltpu.sync_copy(w2_h.at[pl.ds(w2start * C * C, _CH * C)],
                                    w2b.at[pl.ds(0, _CH * C)])

                    @plsc.parallel_loop(0, _CHN, 1)
                    def node(n):
                        nid = n0 + n
                        g1v = gbuf[0, pl.ds(n * C, _LANES)]
                        b1v = gbuf[1, pl.ds(n * C, _LANES)]
                        g2v = gbuf[2, pl.ds(n * C, _LANES)]
                        b2v = gbuf[3, pl.ds(n * C, _LANES)]
                        y = []
                        for c in range(C):
                            z = zbuf[n, c, :]
                            mu, var = _row_stats(z)
                            y.append(_bn_elu_row(z, g1v[c], b1v[c], mu, var))
                        valid = (nid >= nfn0) & (nid < nfn0 + F)
                        vf = jnp.where(valid, 1.0, 0.0)
                        rr = jnp.clip(nid - nfn0 - w2start, 0, _CHN - 1)
                        u = [None] * C
                        for i in range(C):
                            wv = w2b[pl.ds((rr * C + i) * C, _LANES)]
                            for j in range(C):
                                t = y[i] * wv[j]
                                u[j] = t if i == 0 else u[j] + t
                        for j in range(C):
                            uj = u[j] * vf
                            mu2, var2 = _row_stats(uj)
                            zbuf[n, j, :] = _bn_elu_row(uj, g2v[j], b2v[j],
                                                        mu2, var2)
                    pltpu.sync_copy(zbuf, zsh.at[pl.ds(n0, _CHN)])
                return 0
            lax.fori_loop(0, it_count(NNC), phb, 0)
            plsc.subcore_barrier()

            if layer == 0:
                def hcopy(it, _):
                    ch = it * NS + s
                    @pl.when(ch < NEC)
                    def _():
                        e0 = ch * _CH
                        pltpu.sync_copy(x0b_h.at[pl.ds(e0, _CH)], hbuf)
                        pltpu.sync_copy(hbuf, h_h.at[pl.ds(e0, _CH)])
                    return 0
                lax.fori_loop(0, it_count(NEC), hcopy, 0)
                plsc.subcore_barrier()

            def phc(it, _):
                ch = it * NS + s
                @pl.when(ch < NE3C)
                def _():
                    k0 = ch * _CH
                    pltpu.sync_copy(e3s_h.at[pl.ds(k0, _CH)], idxa)
                    pltpu.sync_copy(e3i_h.at[pl.ds(k0, _CH)], idxb)
                    pltpu.sync_copy(w3_h.at[pl.ds(k0 * C, _CH * C)],
                                    wbuf.at[pl.ds(0, _CH * C)])
                    pltpu.async_copy(zsh.at[idxa], stage, sem1).wait()
                    pltpu.async_copy(x0b_h.at[idxb], hbuf, sem2).wait()

                    @plsc.parallel_loop(0, _CH, 1, unroll=4)
                    def edge(i):
                        acc = hbuf[i, :]
                        wv = wbuf[pl.ds(i * C, _LANES)]
                        for c in range(C):
                            acc = acc + stage[i, c, :] * wv[c]
                        hbuf[i, :] = acc
                    pltpu.sync_copy(hbuf, h_h.at[idxb])
                return 0
            lax.fori_loop(0, it_count(NE3C), phc, 0)
            plsc.subcore_barrier()

            if layer < _L - 1:
                def zeroz2(it, _):
                    ch = it * NS + s
                    @pl.when(ch < NZC)
                    def _():
                        pltpu.sync_copy(zbuf, zsh.at[pl.ds(ch * _CHN, _CHN)])
                    return 0
                lax.fori_loop(0, it_count(NZC), zeroz2, 0)
                plsc.subcore_barrier()

        def fin(it, _):
            ch = it * NS + s
            @pl.when(ch < NEC)
            def _():
                e0 = ch * _CH
                pltpu.sync_copy(h_h.at[pl.ds(e0, _CH)], hbuf)
                pltpu.sync_copy(dst_h.at[pl.ds(e0, _CH)], idxa)
                pltpu.sync_copy(hbuf, onsh.at[idxa], add=True)
            return 0
        lax.fori_loop(0, it_count(NEC), fin, 0)
        plsc.subcore_barrier()

        def outcopy(it, _):
            ch = it * NS + s
            @pl.when(ch < NOC)
            def _():
                pltpu.sync_copy(onsh.at[pl.ds(N - _N_OUT + ch * 8, 8)], obuf)
                pltpu.sync_copy(obuf, out_h.at[pl.ds(ch * 8, 8)])
            return 0
        lax.fori_loop(0, it_count(NOC), outcopy, 0)

    out_t, _, _ = gsnn(x_t, src, dst, w1r, w2r, g1, beta1, g2, beta2,
                       w3r_p, e3src_p, e3ids_p, b3, nfn0_vec)
    return out_t.T
